# baseline (speedup 1.0000x reference)
import functools

import jax
import jax.numpy as jnp
from jax import lax
from jax.experimental import pallas as pl
from jax.experimental.pallas import tpu as pltpu
from jax.experimental.pallas import tpu_sc as plsc

_N_ROWS = 16384
_N_COLS = 1204
_NUM_CLASSES = 1203
_LOGIT_THR = 0.8472978603872034
_COMMON_START = 337.0
_FREQ_START = 798.0
_LOG2E = 1.4426950408889634
_LN2 = 0.6931471805599453

_LOG1P_C = (
    3.386965319318591e-08, 0.9999942724811738, -0.4998385618341258,
    0.33154861651921536, -0.23982616049773758, 0.16582275267795007,
    -0.09325203897171261, 0.03484971246998992, -0.006151470959681176,
)

_TC_COLS = 1152
_BLOCK_ROWS = 2048
_COL_BLOCK = 1152

_NW = 16
_ROWS_PER_W = _N_ROWS // _NW
_CHUNK = 128
_NCHUNK = _ROWS_PER_W // _CHUNK
_SC_COL0 = 1152
_SC_W = 52


def _tc_kernel(lbl_full_ref, x_ref, lbl_ref, out_ref, smem, tri_ref):
    r = pl.program_id(0)
    c = pl.program_id(1)
    ncb = pl.num_programs(1)

    @pl.when((r == 0) & (c == 0))
    def _first():
        smem[0] = 0
        smem[1] = jnp.sum((lbl_full_ref[...] == _NUM_CLASSES).astype(jnp.int32))
        ii = jax.lax.broadcasted_iota(jnp.int32, (_BLOCK_ROWS, _BLOCK_ROWS), 0)
        jj = jax.lax.broadcasted_iota(jnp.int32, (_BLOCK_ROWS, _BLOCK_ROWS), 1)
        tri_ref[...] = (jj < ii).astype(jnp.float32)

    lbl = lbl_ref[...]
    bg = lbl == _NUM_CLASSES
    bg_f = bg.astype(jnp.float32)
    prefix = smem[0]
    nb = smem[1]

    @pl.when(c == ncb - 1)
    def _bump():
        smem[0] = prefix + jnp.sum(bg_f.astype(jnp.int32))

    rank = jax.lax.dot_general(
        tri_ref[...], bg_f, (((1,), (0,)), ((), ())),
        preferred_element_type=jnp.float32,
    ) + prefix.astype(jnp.float32)
    t1 = jnp.floor((nb.astype(jnp.float32) + 0.5) * 0.01)
    t2 = jnp.floor((nb.astype(jnp.float32) + 0.5) * 0.1)
    start = jnp.where(rank < t1, 0.0,
                      jnp.where(rank < t2, _COMMON_START, _FREQ_START))

    x = x_ref[...]
    cols = jax.lax.broadcasted_iota(jnp.int32, (1, _COL_BLOCK), 1) + c * _COL_BLOCK
    cols_f = cols.astype(jnp.float32)
    is_lbl = cols == lbl

    x2 = x * _LOG2E
    sp2 = jnp.log2(1.0 + jnp.exp2(x2))

    lhs = jnp.where(bg, cols_f, x)
    rhs = jnp.where(bg, start, _LOGIT_THR)
    base = jnp.where(lhs >= rhs, sp2, 0.0)
    contrib = jnp.where(is_lbl, sp2 - x2, base)

    acc = jnp.sum(contrib, keepdims=True)

    @pl.when((r == 0) & (c == 0))
    def _init():
        out_ref[...] = acc

    @pl.when((r != 0) | (c != 0))
    def _acc():
        out_ref[...] += acc


def _sc_softplus(xv):
    a = jnp.abs(xv)
    t = jnp.exp(-a)
    q = jnp.float32(_LOG1P_C[8])
    for coef in _LOG1P_C[7::-1]:
        q = q * t + jnp.float32(coef)
    return jnp.maximum(xv, 0.0) + q


def _sc_kernel(x_hbm, lbl_hbm, out_hbm, lbl_v, xbuf, outbuf, sem0, sem1):
    wid = lax.axis_index("s")
    base = wid * _ROWS_PER_W

    pltpu.sync_copy(lbl_hbm.at[pl.ds(base, _ROWS_PER_W)],
                    lbl_v.at[pl.ds(0, _ROWS_PER_W)])

    iota_f = lax.iota(jnp.int32, 16).astype(jnp.float32)

    def _row_body(j, acc, g, buf):
        lbl_s = lbl_v[pl.ds(g * _CHUNK + j, 16)][0]
        bg = lbl_s == _NUM_CLASSES
        rhs_s = jnp.where(bg, jnp.float32(-3.0e38), jnp.float32(_LOGIT_THR))
        rhs = jnp.full((16,), rhs_s)
        lbl_f = jnp.full((16,), lbl_s.astype(jnp.float32))
        for k in range(4):
            off = k * 16 if k < 3 else _SC_W - 16
            colv = iota_f + jnp.float32(_SC_COL0 + off)
            xv = xbuf[buf, j, pl.ds(off, 16)]
            sp = _sc_softplus(xv)
            contrib = jnp.where(xv >= rhs, sp, 0.0)
            contrib = jnp.where(colv == lbl_f, sp - xv, contrib)
            if k == 3:
                contrib = jnp.where(colv >= jnp.float32(_SC_COL0 + 48),
                                    contrib, 0.0)
            acc = acc + contrib
        return acc

    def _start_dma(g, buf, sem):
        r0 = base + g * _CHUNK
        return pltpu.async_copy(
            x_hbm.at[pl.ds(r0, _CHUNK), pl.ds(_SC_COL0, _SC_W)],
            xbuf.at[buf], sem)

    sems = (sem0, sem1)
    _start_dma(0, 0, sems[0])
    _start_dma(1, 1, sems[1])

    def _pair_body(p, acc):
        for b in (0, 1):
            g = p * 2 + b
            pltpu.make_async_copy(
                x_hbm.at[pl.ds(0, _CHUNK), pl.ds(_SC_COL0, _SC_W)],
                xbuf.at[b], sems[b]).wait()
            acc = lax.fori_loop(
                0, _CHUNK, functools.partial(_row_body, g=g, buf=b), acc)

            @pl.when(g + 2 < _NCHUNK)
            def _next():
                _start_dma(g + 2, b, sems[b])
        return acc

    acc = lax.fori_loop(
        0, _NCHUNK // 2, _pair_body, jnp.zeros((16,), jnp.float32))

    outbuf[...] = acc
    pltpu.sync_copy(outbuf, out_hbm.at[pl.ds(wid * 16, 16)])


@functools.partial(jax.jit, static_argnames=("interpret",))
def kernel(cls_logits, labels, interpret=False):
    n_i, n_c = cls_logits.shape
    nrb = _N_ROWS // _BLOCK_ROWS
    ncb = _TC_COLS // _COL_BLOCK

    lbl2 = labels.reshape(n_i, 1)

    mesh = plsc.VectorSubcoreMesh(core_axis_name="c", subcore_axis_name="s",
                                  num_cores=1)
    sc_out = pl.kernel(
        _sc_kernel,
        mesh=mesh,
        out_type=jax.ShapeDtypeStruct((_NW * 16,), jnp.float32),
        scratch_types=[
            pltpu.VMEM((_ROWS_PER_W + 16,), jnp.int32),
            pltpu.VMEM((2, _CHUNK, _SC_W), jnp.float32),
            pltpu.VMEM((16,), jnp.float32),
            pltpu.SemaphoreType.DMA,
            pltpu.SemaphoreType.DMA,
        ],
    )(cls_logits, labels)

    tc_out = pl.pallas_call(
        _tc_kernel,
        grid=(nrb, ncb),
        in_specs=[
            pl.BlockSpec((128, 128), lambda r, c: (0, 0)),
            pl.BlockSpec((_BLOCK_ROWS, _COL_BLOCK), lambda r, c: (r, c)),
            pl.BlockSpec((_BLOCK_ROWS, 1), lambda r, c: (r, 0)),
        ],
        out_specs=pl.BlockSpec((1, 1), lambda r, c: (0, 0)),
        out_shape=jax.ShapeDtypeStruct((1, 1), cls_logits.dtype),
        scratch_shapes=[
            pltpu.SMEM((2,), jnp.int32),
            pltpu.VMEM((_BLOCK_ROWS, _BLOCK_ROWS), jnp.float32),
        ],
        interpret=interpret,
    )(labels.reshape(128, 128), cls_logits, lbl2)

    return (tc_out[0, 0] * _LN2 + jnp.sum(sc_out)) * (1.0 / _N_ROWS)


# speedup vs baseline: 1.0716x; 1.0716x over previous
import functools

import jax
import jax.numpy as jnp
from jax import lax
from jax.experimental import pallas as pl
from jax.experimental.pallas import tpu as pltpu
from jax.experimental.pallas import tpu_sc as plsc

_N_ROWS = 16384
_N_COLS = 1204
_NUM_CLASSES = 1203
_LOGIT_THR = 0.8472978603872034
_COMMON_START = 337.0
_FREQ_START = 798.0
_LOG2E = 1.4426950408889634
_LN2 = 0.6931471805599453

_LOG1P_C = (
    3.386965319318591e-08, 0.9999942724811738, -0.4998385618341258,
    0.33154861651921536, -0.23982616049773758, 0.16582275267795007,
    -0.09325203897171261, 0.03484971246998992, -0.006151470959681176,
)

_TC_COLS = 1152
_BLOCK_ROWS = 1024
_COL_BLOCK = 1152

_NW = 16
_ROWS_PER_W = _N_ROWS // _NW
_CHUNK = 128
_NCHUNK = _ROWS_PER_W // _CHUNK
_SC_COL0 = 1152
_SC_W = 52


def _tc_kernel(lbl_full_ref, x_ref, lbl_ref, out_ref, smem, tri_ref):
    r = pl.program_id(0)
    c = pl.program_id(1)
    ncb = pl.num_programs(1)

    @pl.when((r == 0) & (c == 0))
    def _first():
        smem[0] = 0
        smem[1] = jnp.sum((lbl_full_ref[...] == _NUM_CLASSES).astype(jnp.int32))
        ii = jax.lax.broadcasted_iota(jnp.int32, (_BLOCK_ROWS, _BLOCK_ROWS), 0)
        jj = jax.lax.broadcasted_iota(jnp.int32, (_BLOCK_ROWS, _BLOCK_ROWS), 1)
        tri_ref[...] = (jj < ii).astype(jnp.float32)

    lbl = lbl_ref[...]
    bg = lbl == _NUM_CLASSES
    bg_f = bg.astype(jnp.float32)
    prefix = smem[0]
    nb = smem[1]

    @pl.when(c == ncb - 1)
    def _bump():
        smem[0] = prefix + jnp.sum(bg_f.astype(jnp.int32))

    rank = jax.lax.dot_general(
        tri_ref[...], bg_f, (((1,), (0,)), ((), ())),
        preferred_element_type=jnp.float32,
    ) + prefix.astype(jnp.float32)
    t1 = jnp.floor((nb.astype(jnp.float32) + 0.5) * 0.01)
    t2 = jnp.floor((nb.astype(jnp.float32) + 0.5) * 0.1)
    start = jnp.where(rank < t1, 0.0,
                      jnp.where(rank < t2, _COMMON_START, _FREQ_START))

    x = x_ref[...]
    cols = jax.lax.broadcasted_iota(jnp.int32, (1, _COL_BLOCK), 1) + c * _COL_BLOCK
    cols_f = cols.astype(jnp.float32)
    is_lbl = cols == lbl

    x2 = x * _LOG2E
    sp2 = jnp.log2(1.0 + jnp.exp2(x2))

    lhs = jnp.where(bg, cols_f, x)
    rhs = jnp.where(bg, start, _LOGIT_THR)
    base = jnp.where(lhs >= rhs, sp2, 0.0)
    contrib = jnp.where(is_lbl, sp2 - x2, base)

    acc = jnp.sum(contrib, keepdims=True)

    @pl.when((r == 0) & (c == 0))
    def _init():
        out_ref[...] = acc

    @pl.when((r != 0) | (c != 0))
    def _acc():
        out_ref[...] += acc


def _sc_softplus(xv):
    a = jnp.abs(xv)
    t = jnp.exp(-a)
    q = jnp.float32(_LOG1P_C[8])
    for coef in _LOG1P_C[7::-1]:
        q = q * t + jnp.float32(coef)
    return jnp.maximum(xv, 0.0) + q


def _sc_kernel(x_hbm, lbl_hbm, out_hbm, lbl_v, xbuf, outbuf, sem0, sem1):
    wid = lax.axis_index("s")
    base = wid * _ROWS_PER_W

    pltpu.sync_copy(lbl_hbm.at[pl.ds(base, _ROWS_PER_W)],
                    lbl_v.at[pl.ds(0, _ROWS_PER_W)])

    iota_f = lax.iota(jnp.int32, 16).astype(jnp.float32)

    def _row_body(j, acc, g, buf):
        lbl_s = lbl_v[pl.ds(g * _CHUNK + j, 16)][0]
        bg = lbl_s == _NUM_CLASSES
        rhs_s = jnp.where(bg, jnp.float32(-3.0e38), jnp.float32(_LOGIT_THR))
        rhs = jnp.full((16,), rhs_s)
        lbl_f = jnp.full((16,), lbl_s.astype(jnp.float32))
        for k in range(4):
            off = k * 16 if k < 3 else _SC_W - 16
            colv = iota_f + jnp.float32(_SC_COL0 + off)
            xv = xbuf[buf, j, pl.ds(off, 16)]
            sp = _sc_softplus(xv)
            contrib = jnp.where(xv >= rhs, sp, 0.0)
            contrib = jnp.where(colv == lbl_f, sp - xv, contrib)
            if k == 3:
                contrib = jnp.where(colv >= jnp.float32(_SC_COL0 + 48),
                                    contrib, 0.0)
            acc = acc + contrib
        return acc

    def _start_dma(g, buf, sem):
        r0 = base + g * _CHUNK
        return pltpu.async_copy(
            x_hbm.at[pl.ds(r0, _CHUNK), pl.ds(_SC_COL0, _SC_W)],
            xbuf.at[buf], sem)

    sems = (sem0, sem1)
    _start_dma(0, 0, sems[0])
    _start_dma(1, 1, sems[1])

    def _pair_body(p, acc):
        for b in (0, 1):
            g = p * 2 + b
            pltpu.make_async_copy(
                x_hbm.at[pl.ds(0, _CHUNK), pl.ds(_SC_COL0, _SC_W)],
                xbuf.at[b], sems[b]).wait()
            acc = lax.fori_loop(
                0, _CHUNK, functools.partial(_row_body, g=g, buf=b), acc)

            @pl.when(g + 2 < _NCHUNK)
            def _next():
                _start_dma(g + 2, b, sems[b])
        return acc

    acc = lax.fori_loop(
        0, _NCHUNK // 2, _pair_body, jnp.zeros((16,), jnp.float32))

    outbuf[...] = acc
    pltpu.sync_copy(outbuf, out_hbm.at[pl.ds(wid * 16, 16)])


@functools.partial(jax.jit, static_argnames=("interpret",))
def kernel(cls_logits, labels, interpret=False):
    n_i, n_c = cls_logits.shape
    nrb = _N_ROWS // _BLOCK_ROWS
    ncb = _TC_COLS // _COL_BLOCK

    lbl2 = labels.reshape(n_i, 1)

    mesh = plsc.VectorSubcoreMesh(core_axis_name="c", subcore_axis_name="s",
                                  num_cores=1)
    sc_out = pl.kernel(
        _sc_kernel,
        mesh=mesh,
        out_type=jax.ShapeDtypeStruct((_NW * 16,), jnp.float32),
        scratch_types=[
            pltpu.VMEM((_ROWS_PER_W + 16,), jnp.int32),
            pltpu.VMEM((2, _CHUNK, _SC_W), jnp.float32),
            pltpu.VMEM((16,), jnp.float32),
            pltpu.SemaphoreType.DMA,
            pltpu.SemaphoreType.DMA,
        ],
    )(cls_logits, labels)

    tc_out = pl.pallas_call(
        _tc_kernel,
        grid=(nrb, ncb),
        in_specs=[
            pl.BlockSpec((128, 128), lambda r, c: (0, 0)),
            pl.BlockSpec((_BLOCK_ROWS, _COL_BLOCK), lambda r, c: (r, c)),
            pl.BlockSpec((_BLOCK_ROWS, 1), lambda r, c: (r, 0)),
        ],
        out_specs=pl.BlockSpec((1, 1), lambda r, c: (0, 0)),
        out_shape=jax.ShapeDtypeStruct((1, 1), cls_logits.dtype),
        scratch_shapes=[
            pltpu.SMEM((2,), jnp.int32),
            pltpu.VMEM((_BLOCK_ROWS, _BLOCK_ROWS), jnp.float32),
        ],
        interpret=interpret,
    )(labels.reshape(128, 128), cls_logits, lbl2)

    return (tc_out[0, 0] * _LN2 + jnp.sum(sc_out)) * (1.0 / _N_ROWS)
